# baseline (speedup 1.0000x reference)
import math

import jax
import jax.numpy as jnp
from jax.experimental import pallas as pl
from jax.experimental.pallas import tpu as pltpu

_N = 16384
_D = 128
_CHUNK = 64
_UNROLL = 64
_ROWS = _CHUNK * _UNROLL


def _pe_kernel(x_ref, f_ref, p_ref, b_ref, out_ref):
    freqs = f_ref[...]
    p = p_ref[...]
    bias = b_ref[...]
    for u in range(_UNROLL):
        rows = pl.ds(u * _CHUNK, _CHUNK)
        arg = x_ref[rows, :] * freqs
        s = jnp.sin(arg)
        c = jnp.cos(arg)
        sc = jnp.concatenate([s, c], axis=1)
        out = jax.lax.dot_general(
            sc,
            p,
            (((1,), (0,)), ((), ())),
            preferred_element_type=jnp.float32,
            precision=jax.lax.Precision.HIGHEST,
        )
        out_ref[rows, :] = out + bias


def kernel(x, E_class):
    del E_class
    x2 = x.reshape(_N, 1)
    angles = jnp.arange(_D, dtype=jnp.float32)
    freqs = (jnp.power(2.0, angles) * math.pi).reshape(1, _D)
    freqs = freqs.at[0, _D - 1].set(0.0)
    k = jnp.arange(_D)
    src = jnp.zeros((2 * _D,), jnp.int32).at[2 * k].set(k).at[2 * k + 1].set(_D + k)
    perm = jax.nn.one_hot(src, 2 * _D, axis=0, dtype=jnp.float32)
    bias = jnp.zeros((1, 2 * _D), jnp.float32).at[0, 2 * _D - 2 :].set(jnp.nan)
    grid = (_N // _ROWS,)
    return pl.pallas_call(
        _pe_kernel,
        grid=grid,
        in_specs=[
            pl.BlockSpec((_ROWS, 1), lambda i: (i, 0)),
            pl.BlockSpec((1, _D), lambda i: (0, 0)),
            pl.BlockSpec((2 * _D, 2 * _D), lambda i: (0, 0)),
            pl.BlockSpec((1, 2 * _D), lambda i: (0, 0)),
        ],
        out_specs=pl.BlockSpec((_ROWS, 2 * _D), lambda i: (i, 0)),
        out_shape=jax.ShapeDtypeStruct((_N, 2 * _D), jnp.float32),
        compiler_params=pltpu.CompilerParams(
            dimension_semantics=("parallel",),
        ),
    )(x2, freqs, perm, bias)


# speedup vs baseline: 1.1365x; 1.1365x over previous
import math

import jax
import jax.numpy as jnp
from jax.experimental import pallas as pl
from jax.experimental.pallas import tpu as pltpu

_N = 16384
_D = 128
_CHUNK = 128
_UNROLL = 32
_ROWS = _CHUNK * _UNROLL


def _pe_kernel(x_ref, f_ref, p_ref, b_ref, out_ref):
    freqs = f_ref[...]
    p = p_ref[...]
    bias = b_ref[...]
    for u in range(_UNROLL):
        rows = pl.ds(u * _CHUNK, _CHUNK)
        arg = x_ref[rows, :] * freqs
        s = jnp.sin(arg)
        c = jnp.cos(arg)
        sc = jnp.concatenate([s, c], axis=1)
        out = jax.lax.dot_general(
            sc,
            p,
            (((1,), (0,)), ((), ())),
            preferred_element_type=jnp.float32,
            precision=jax.lax.Precision.HIGHEST,
        )
        out_ref[rows, :] = out + bias


def kernel(x, E_class):
    del E_class
    x2 = x.reshape(_N, 1)
    angles = jnp.arange(_D, dtype=jnp.float32)
    freqs = (jnp.power(2.0, angles) * math.pi).reshape(1, _D)
    freqs = freqs.at[0, _D - 1].set(0.0)
    k = jnp.arange(_D)
    src = jnp.zeros((2 * _D,), jnp.int32).at[2 * k].set(k).at[2 * k + 1].set(_D + k)
    perm = jax.nn.one_hot(src, 2 * _D, axis=0, dtype=jnp.float32)
    bias = jnp.zeros((1, 2 * _D), jnp.float32).at[0, 2 * _D - 2 :].set(jnp.nan)
    grid = (_N // _ROWS,)
    return pl.pallas_call(
        _pe_kernel,
        grid=grid,
        in_specs=[
            pl.BlockSpec((_ROWS, 1), lambda i: (i, 0)),
            pl.BlockSpec((1, _D), lambda i: (0, 0)),
            pl.BlockSpec((2 * _D, 2 * _D), lambda i: (0, 0)),
            pl.BlockSpec((1, 2 * _D), lambda i: (0, 0)),
        ],
        out_specs=pl.BlockSpec((_ROWS, 2 * _D), lambda i: (i, 0)),
        out_shape=jax.ShapeDtypeStruct((_N, 2 * _D), jnp.float32),
        compiler_params=pltpu.CompilerParams(
            dimension_semantics=("parallel",),
        ),
    )(x2, freqs, perm, bias)


# speedup vs baseline: 1.2637x; 1.1120x over previous
import math

import jax
import jax.numpy as jnp
import numpy as np
from jax.experimental import pallas as pl
from jax.experimental.pallas import tpu as pltpu

_N = 16384
_D = 128
_CHUNK = 128
_UNROLL = 32
_ROWS = _CHUNK * _UNROLL


def _pe_kernel(x_ref, f_ref, p_ref, b_ref, out_ref):
    freqs = f_ref[...]
    p = p_ref[...]
    bias = b_ref[...]
    for u in range(_UNROLL):
        rows = pl.ds(u * _CHUNK, _CHUNK)
        arg = x_ref[rows, :] * freqs
        s = jnp.sin(arg)
        c = jnp.cos(arg)
        sc = jnp.concatenate([s, c], axis=1)
        out = jax.lax.dot_general(
            sc,
            p,
            (((1,), (0,)), ((), ())),
            preferred_element_type=jnp.float32,
            precision=jax.lax.Precision.HIGHEST,
        )
        out_ref[rows, :] = out + bias


def kernel(x, E_class):
    del E_class
    x2 = x.reshape(_N, 1)
    angles = jnp.arange(_D, dtype=jnp.float32)
    freqs = (jnp.power(2.0, angles) * math.pi).reshape(1, _D)
    freqs = freqs.at[0, _D - 1].set(0.0)
    perm = np.zeros((2 * _D, 2 * _D), np.float32)
    kk = np.arange(_D)
    perm[kk, 2 * kk] = 1.0
    perm[_D + kk, 2 * kk + 1] = 1.0
    bias = np.zeros((1, 2 * _D), np.float32)
    bias[0, 2 * _D - 2 :] = np.nan
    grid = (_N // _ROWS,)
    return pl.pallas_call(
        _pe_kernel,
        grid=grid,
        in_specs=[
            pl.BlockSpec((_ROWS, 1), lambda i: (i, 0)),
            pl.BlockSpec((1, _D), lambda i: (0, 0)),
            pl.BlockSpec((2 * _D, 2 * _D), lambda i: (0, 0)),
            pl.BlockSpec((1, 2 * _D), lambda i: (0, 0)),
        ],
        out_specs=pl.BlockSpec((_ROWS, 2 * _D), lambda i: (i, 0)),
        out_shape=jax.ShapeDtypeStruct((_N, 2 * _D), jnp.float32),
        compiler_params=pltpu.CompilerParams(
            dimension_semantics=("parallel",),
        ),
    )(x2, freqs, perm, bias)


# speedup vs baseline: 1.3859x; 1.0967x over previous
import math

import jax
import jax.numpy as jnp
import numpy as np
from jax.experimental import pallas as pl
from jax.experimental.pallas import tpu as pltpu

_N = 16384
_D = 128
_UNROLL = 32
_XROWS = _N // _D


def _pe_kernel(x_ref, f_ref, p_ref, b_ref, out_ref):
    fcol = f_ref[...]
    p = p_ref[...]
    bias = b_ref[...]
    for u in range(_UNROLL):
        xrow = x_ref[pl.ds(u, 1), :]
        arg = fcol * xrow
        s = jnp.sin(arg)
        c = jnp.cos(arg)
        sc = jnp.concatenate([s, c], axis=0)
        out = jax.lax.dot_general(
            sc,
            p,
            (((0,), (0,)), ((), ())),
            preferred_element_type=jnp.float32,
            precision=jax.lax.Precision.HIGHEST,
        )
        out_ref[pl.ds(u * _D, _D), :] = out + bias


def kernel(x, E_class):
    del E_class
    x2 = x.reshape(_XROWS, _D)
    angles = jnp.arange(_D, dtype=jnp.float32)
    freqs = (jnp.power(2.0, angles) * math.pi).reshape(_D, 1)
    freqs = freqs.at[_D - 1, 0].set(0.0)
    perm = np.zeros((2 * _D, 2 * _D), np.float32)
    kk = np.arange(_D)
    perm[kk, 2 * kk] = 1.0
    perm[_D + kk, 2 * kk + 1] = 1.0
    bias = np.zeros((1, 2 * _D), np.float32)
    bias[0, 2 * _D - 2 :] = np.nan
    grid = (_XROWS // _UNROLL,)
    return pl.pallas_call(
        _pe_kernel,
        grid=grid,
        in_specs=[
            pl.BlockSpec((_UNROLL, _D), lambda i: (i, 0)),
            pl.BlockSpec((_D, 1), lambda i: (0, 0)),
            pl.BlockSpec((2 * _D, 2 * _D), lambda i: (0, 0)),
            pl.BlockSpec((1, 2 * _D), lambda i: (0, 0)),
        ],
        out_specs=pl.BlockSpec((_UNROLL * _D, 2 * _D), lambda i: (i, 0)),
        out_shape=jax.ShapeDtypeStruct((_N, 2 * _D), jnp.float32),
        compiler_params=pltpu.CompilerParams(
            dimension_semantics=("parallel",),
        ),
    )(x2, freqs, perm, bias)
